# initial kernel scaffold (unmeasured)
import jax
import jax.numpy as jnp
from jax import lax
from jax.experimental import pallas as pl
from jax.experimental.pallas import tpu as pltpu

N = 16
SQ = 256
D = 1024
H = 8
DH = 128
SKV = 4096
SCALE = 0.08838834764831843


def _body(x_ref, wq_ref, wo_ref, k_ref, v_ref, out_ref,
          xfull, pbuf, rsbuf, stage, qbuf, obuf,
          ag_send, ag_recv, rs_send, rs_recv):
    i = lax.axis_index("i")
    left = (i - 1) % N
    right = (i + 1) % N

    barrier = pltpu.get_barrier_semaphore()
    for nbr in (left, right):
        pl.semaphore_signal(barrier, inc=1, device_id=(nbr,),
                            device_id_type=pl.DeviceIdType.MESH)
    pl.semaphore_wait(barrier, 2)

    xfull[i] = x_ref[...]

    for h in range(N - 1):
        blk_out = (i - h) % N
        blk_in = (i - h - 1) % N
        send = pltpu.make_async_remote_copy(
            src_ref=xfull.at[blk_out], dst_ref=xfull.at[blk_out],
            send_sem=ag_send.at[h], recv_sem=ag_recv.at[h],
            device_id=(right,), device_id_type=pl.DeviceIdType.MESH)
        send.start()
        recv = pltpu.make_async_remote_copy(
            src_ref=xfull.at[blk_in], dst_ref=xfull.at[blk_in],
            send_sem=ag_send.at[h], recv_sem=ag_recv.at[h],
            device_id=(right,), device_id_type=pl.DeviceIdType.MESH)
        send.wait_send()
        recv.wait_recv()

    def chunk(c, carry):
        xc = xfull[c]
        q = jnp.dot(xc, wq_ref[...], preferred_element_type=jnp.float32)
        qbuf[...] = (q * SCALE).astype(jnp.bfloat16)
        for h in range(H):
            qh = qbuf[:, h * DH:(h + 1) * DH]
            s = lax.dot_general(qh, k_ref[h], (((1,), (1,)), ((), ())),
                                preferred_element_type=jnp.float32)
            p = jnp.exp(s)
            l = jnp.sum(p, axis=1, keepdims=True)
            o = jnp.dot(p.astype(jnp.bfloat16), v_ref[h],
                        preferred_element_type=jnp.float32)
            obuf[:, h * DH:(h + 1) * DH] = (o / l).astype(jnp.bfloat16)
        pc = jnp.dot(obuf[...], wo_ref[...], preferred_element_type=jnp.float32)
        pbuf[c] = pc.astype(jnp.bfloat16)
        return carry

    lax.fori_loop(0, N, chunk, 0)

    stage[...] = pbuf[(i - 1) % N].astype(jnp.float32)
    for k in range(N - 1):
        rdma = pltpu.make_async_remote_copy(
            src_ref=stage, dst_ref=rsbuf.at[k],
            send_sem=rs_send.at[k], recv_sem=rs_recv.at[k],
            device_id=(right,), device_id_type=pl.DeviceIdType.MESH)
        rdma.start()
        rdma.wait()
        if k < N - 2:
            stage[...] = rsbuf[k] + pbuf[(i - 2 - k) % N].astype(jnp.float32)
    out_ref[...] = rsbuf[N - 2] + pbuf[i].astype(jnp.float32)


def kernel(x, Wq, Wo, K_ext, V_ext):
    i = lax.axis_index("i")
    xb = x[0].astype(jnp.bfloat16)
    wq = Wq.astype(jnp.bfloat16)
    wo = Wo.astype(jnp.bfloat16)
    k = lax.dynamic_slice_in_dim(K_ext[0], i * H, H, axis=1)
    v = lax.dynamic_slice_in_dim(V_ext[0], i * H, H, axis=1)
    k = jnp.transpose(k, (1, 0, 2)).astype(jnp.bfloat16)
    v = jnp.transpose(v, (1, 0, 2)).astype(jnp.bfloat16)

    out = pl.pallas_call(
        _body,
        out_shape=jax.ShapeDtypeStruct((SQ, D), jnp.float32),
        in_specs=[pl.BlockSpec(memory_space=pltpu.VMEM)] * 5,
        out_specs=pl.BlockSpec(memory_space=pltpu.VMEM),
        scratch_shapes=[
            pltpu.VMEM((N, SQ, D), jnp.bfloat16),
            pltpu.VMEM((N, SQ, D), jnp.bfloat16),
            pltpu.VMEM((N - 1, SQ, D), jnp.float32),
            pltpu.VMEM((SQ, D), jnp.float32),
            pltpu.VMEM((SQ, D), jnp.bfloat16),
            pltpu.VMEM((SQ, D), jnp.bfloat16),
            pltpu.SemaphoreType.DMA((N - 1,)),
            pltpu.SemaphoreType.DMA((N - 1,)),
            pltpu.SemaphoreType.DMA((N - 1,)),
            pltpu.SemaphoreType.DMA((N - 1,)),
        ],
        compiler_params=pltpu.CompilerParams(collective_id=0),
    )(xb, wq, wo, k, v)
    return out[None]


# baseline (device time: 523548 ns/iter reference)
import jax
import jax.numpy as jnp
from jax import lax
from jax.experimental import pallas as pl
from jax.experimental.pallas import tpu as pltpu

N = 16
SQ = 256
D = 1024
H = 8
DH = 128
SKV = 4096
SCALE = 0.08838834764831843


def _body(x_ref, wq_ref, wo_ref, k_ref, v_ref, out_ref,
          xfull, pbuf, rsbuf, stage, qbuf, obuf,
          ag_send, ag_recv, rs_send, rs_recv):
    i = lax.axis_index("i")
    left = (i - 1) % N
    right = (i + 1) % N

    barrier = pltpu.get_barrier_semaphore()
    for nbr in (left, right):
        pl.semaphore_signal(barrier, inc=1, device_id=(nbr,),
                            device_id_type=pl.DeviceIdType.MESH)
    pl.semaphore_wait(barrier, 2)

    xfull[i] = x_ref[...]

    for h in range(N - 1):
        blk_out = (i - h) % N
        blk_in = (i - h - 1) % N
        send = pltpu.make_async_remote_copy(
            src_ref=xfull.at[blk_out], dst_ref=xfull.at[blk_out],
            send_sem=ag_send.at[h], recv_sem=ag_recv.at[h],
            device_id=(right,), device_id_type=pl.DeviceIdType.MESH)
        send.start()
        recv = pltpu.make_async_remote_copy(
            src_ref=xfull.at[blk_in], dst_ref=xfull.at[blk_in],
            send_sem=ag_send.at[h], recv_sem=ag_recv.at[h],
            device_id=(right,), device_id_type=pl.DeviceIdType.MESH)
        send.wait_send()
        recv.wait_recv()

    def chunk(c, carry):
        xc = xfull[c]
        q = jnp.dot(xc, wq_ref[...], preferred_element_type=jnp.float32)
        qbuf[...] = (q * SCALE).astype(jnp.bfloat16)
        for h in range(H):
            qh = qbuf[:, h * DH:(h + 1) * DH]
            s = lax.dot_general(qh, k_ref[h], (((1,), (1,)), ((), ())),
                                preferred_element_type=jnp.float32)
            p = jnp.exp(s)
            l = jnp.sum(p, axis=1, keepdims=True)
            o = jnp.dot(p.astype(jnp.bfloat16), v_ref[h],
                        preferred_element_type=jnp.float32)
            obuf[:, h * DH:(h + 1) * DH] = (o / l).astype(jnp.bfloat16)
        pc = jnp.dot(obuf[...], wo_ref[...], preferred_element_type=jnp.float32)
        pbuf[c] = pc.astype(jnp.bfloat16)
        return carry

    lax.fori_loop(0, N, chunk, 0)

    stage[...] = pbuf[(i - 1) % N].astype(jnp.float32)
    for k in range(N - 1):
        rdma = pltpu.make_async_remote_copy(
            src_ref=stage, dst_ref=rsbuf.at[k],
            send_sem=rs_send.at[k], recv_sem=rs_recv.at[k],
            device_id=(right,), device_id_type=pl.DeviceIdType.MESH)
        rdma.start()
        rdma.wait()
        if k < N - 2:
            stage[...] = rsbuf[k] + pbuf[(i - 2 - k) % N].astype(jnp.float32)
    out_ref[...] = rsbuf[N - 2] + pbuf[i].astype(jnp.float32)


def kernel(x, Wq, Wo, K_ext, V_ext):
    i = lax.axis_index("i")
    xb = x[0].astype(jnp.bfloat16)
    wq = Wq.astype(jnp.bfloat16)
    wo = Wo.astype(jnp.bfloat16)
    k = lax.dynamic_slice_in_dim(K_ext[0], i * H, H, axis=1)
    v = lax.dynamic_slice_in_dim(V_ext[0], i * H, H, axis=1)
    k = jnp.transpose(k, (1, 0, 2)).astype(jnp.bfloat16)
    v = jnp.transpose(v, (1, 0, 2)).astype(jnp.bfloat16)

    out = pl.pallas_call(
        _body,
        out_shape=jax.ShapeDtypeStruct((SQ, D), jnp.float32),
        in_specs=[pl.BlockSpec(memory_space=pltpu.VMEM)] * 5,
        out_specs=pl.BlockSpec(memory_space=pltpu.VMEM),
        scratch_shapes=[
            pltpu.VMEM((N, SQ, D), jnp.bfloat16),
            pltpu.VMEM((N, SQ, D), jnp.bfloat16),
            pltpu.VMEM((N - 1, SQ, D), jnp.float32),
            pltpu.VMEM((SQ, D), jnp.float32),
            pltpu.VMEM((SQ, D), jnp.bfloat16),
            pltpu.VMEM((SQ, D), jnp.bfloat16),
            pltpu.SemaphoreType.DMA((N - 1,)),
            pltpu.SemaphoreType.DMA((N - 1,)),
            pltpu.SemaphoreType.DMA((N - 1,)),
            pltpu.SemaphoreType.DMA((N - 1,)),
        ],
        compiler_params=pltpu.CompilerParams(
            collective_id=0, vmem_limit_bytes=100 * 1024 * 1024),
    )(xb, wq, wo, k, v)
    return out[None]


# device time: 245006 ns/iter; 2.1369x vs baseline; 2.1369x over previous
import jax
import jax.numpy as jnp
from jax import lax
from jax.experimental import pallas as pl
from jax.experimental.pallas import tpu as pltpu

N = 16
SQ = 256
D = 1024
H = 8
DH = 128
SKV = 4096
SCALE = 0.08838834764831843


def _body(x_ref, wq_ref, wo_ref, k_ref, v_ref, out_ref,
          xfull, pbuf, rsbuf, stage, qbuf, obuf,
          ag_send, ag_recv, rs_send, rs_recv):
    i = lax.axis_index("i")
    left = (i - 1) % N
    right = (i + 1) % N

    barrier = pltpu.get_barrier_semaphore()
    for nbr in (left, right):
        pl.semaphore_signal(barrier, inc=1, device_id=(nbr,),
                            device_id_type=pl.DeviceIdType.MESH)
    pl.semaphore_wait(barrier, 2)

    xfull[i] = x_ref[...]

    def ag_desc(h, blk):
        return pltpu.make_async_remote_copy(
            src_ref=xfull.at[blk], dst_ref=xfull.at[blk],
            send_sem=ag_send.at[h], recv_sem=ag_recv.at[h],
            device_id=(right,), device_id_type=pl.DeviceIdType.MESH)

    def rs_desc(j, slot):
        return pltpu.make_async_remote_copy(
            src_ref=stage.at[slot], dst_ref=rsbuf.at[j],
            send_sem=rs_send.at[j], recv_sem=rs_recv.at[j],
            device_id=(right,), device_id_type=pl.DeviceIdType.MESH)

    def compute_chunk(c):
        xc = xfull[c]
        q = jnp.dot(xc, wq_ref[...], preferred_element_type=jnp.float32)
        qbuf[...] = (q * SCALE).astype(jnp.bfloat16)
        for h in range(H):
            qh = qbuf[:, h * DH:(h + 1) * DH]
            s = lax.dot_general(qh, k_ref[h], (((1,), (1,)), ((), ())),
                                preferred_element_type=jnp.float32)
            p = jnp.exp(s)
            l = jnp.sum(p, axis=1, keepdims=True)
            o = jnp.dot(p.astype(jnp.bfloat16), v_ref[h],
                        preferred_element_type=jnp.float32)
            obuf[:, h * DH:(h + 1) * DH] = (o / l).astype(jnp.bfloat16)
        pc = jnp.dot(obuf[...], wo_ref[...], preferred_element_type=jnp.float32)
        pbuf[c] = pc.astype(jnp.bfloat16)

    ag_desc(0, i).start()
    compute_chunk(i)

    c0 = (i - 1) % N
    ag_desc(0, c0).wait_recv()
    ag_desc(1, c0).start()
    compute_chunk(c0)
    stage[0] = pbuf[c0]
    rs_desc(0, 0).start()

    def step(j, carry):
        c = (i - 1 - j) % N
        ag_desc(j, c).wait_recv()

        @pl.when(j < N - 2)
        def _():
            ag_desc(j + 1, c).start()

        compute_chunk(c)
        rs_desc(j - 1, 0).wait_recv()

        @pl.when(j >= 2)
        def _():
            jj = jnp.maximum(j - 2, 0)
            rs_desc(jj, jj % 2).wait_send()

        stage[j % 2] = (rsbuf[j - 1].astype(jnp.float32)
                        + pbuf[c].astype(jnp.float32)).astype(jnp.bfloat16)
        rs_desc(j, j % 2).start()
        return carry

    lax.fori_loop(1, N - 1, step, 0)

    rs_desc(N - 2, 0).wait_recv()
    out_ref[...] = rsbuf[N - 2].astype(jnp.float32) + pbuf[i].astype(jnp.float32)

    def drain(h, carry):
        ag_desc(h, (i - h) % N).wait_send()
        return carry

    lax.fori_loop(0, N - 1, drain, 0)
    rs_desc(N - 3, (N - 3) % 2).wait_send()
    rs_desc(N - 2, (N - 2) % 2).wait_send()


def kernel(x, Wq, Wo, K_ext, V_ext):
    i = lax.axis_index("i")
    xb = x[0].astype(jnp.bfloat16)
    wq = Wq.astype(jnp.bfloat16)
    wo = Wo.astype(jnp.bfloat16)
    k = lax.dynamic_slice_in_dim(K_ext[0], i * H, H, axis=1)
    v = lax.dynamic_slice_in_dim(V_ext[0], i * H, H, axis=1)
    k = jnp.transpose(k, (1, 0, 2)).astype(jnp.bfloat16)
    v = jnp.transpose(v, (1, 0, 2)).astype(jnp.bfloat16)

    out = pl.pallas_call(
        _body,
        out_shape=jax.ShapeDtypeStruct((SQ, D), jnp.float32),
        in_specs=[pl.BlockSpec(memory_space=pltpu.VMEM)] * 5,
        out_specs=pl.BlockSpec(memory_space=pltpu.VMEM),
        scratch_shapes=[
            pltpu.VMEM((N, SQ, D), jnp.bfloat16),
            pltpu.VMEM((N, SQ, D), jnp.bfloat16),
            pltpu.VMEM((N - 1, SQ, D), jnp.bfloat16),
            pltpu.VMEM((2, SQ, D), jnp.bfloat16),
            pltpu.VMEM((SQ, D), jnp.bfloat16),
            pltpu.VMEM((SQ, D), jnp.bfloat16),
            pltpu.SemaphoreType.DMA((N,)),
            pltpu.SemaphoreType.DMA((N,)),
            pltpu.SemaphoreType.DMA((N,)),
            pltpu.SemaphoreType.DMA((N,)),
        ],
        compiler_params=pltpu.CompilerParams(
            collective_id=0, vmem_limit_bytes=100 * 1024 * 1024),
    )(xb, wq, wo, k, v)
    return out[None]
